# baseline (speedup 1.0000x reference)
import functools
import jax
import jax.numpy as jnp
from jax import lax
from jax.experimental import pallas as pl
from jax.experimental.pallas import tpu as pltpu
from jax.experimental.pallas import tpu_sc as plsc

VOCAB = 30522
HIDDEN = 768
MAX_POS = 512
TYPE_VOCAB = 16
BATCH = 128
SEQ = 512

L = 16
NW = 32
PBLK = SEQ // NW
SCH = 4
CHT = SCH * PBLK
NCHUNK = BATCH // SCH
PAD = HIDDEN + 1
NG = HIDDEN // L
UNROLL = 8
INV_H = 1.0 / HIDDEN
EPS = 1e-12

_DNUMS = lax.GatherDimensionNumbers(
    offset_dims=(), collapsed_slice_dims=(0,), start_index_map=(0,))


def _splat(vec, j):
    return lax.gather(vec, jnp.full((L, 1), j, jnp.int32), _DNUMS, (1,),
                      mode=lax.GatherScatterMode.PROMISE_IN_BOUNDS)


def _rsqrt(x):
    xi = plsc.bitcast(x, jnp.int32)
    yi = jnp.int32(0x5F3759DF) - (xi >> 1)
    y = plsc.bitcast(yi, jnp.float32)
    for _ in range(3):
        y = y * (1.5 - 0.5 * x * y * y)
    return y


def _body(ids_hbm, tt_hbm, word_hbm, pos_hbm, type_hbm, gamma_hbm, beta_hbm,
          out_hbm, idx_v, tt_v, rows_v, pos_v, type_v, gam_v, bet_v, sem):
    wid = lax.axis_index("s") * 2 + lax.axis_index("c")
    p0 = wid * PBLK

    pltpu.sync_copy(pos_hbm.at[pl.ds(p0, PBLK), :], pos_v.at[:, pl.ds(0, HIDDEN)])
    pltpu.sync_copy(type_hbm, type_v.at[:, pl.ds(0, HIDDEN)])
    pltpu.sync_copy(gamma_hbm, gam_v)
    pltpu.sync_copy(beta_hbm, bet_v)

    lanes = lax.iota(jnp.int32, L)
    zero = jnp.zeros((L,), jnp.float32)
    zero_i = jnp.zeros((L,), jnp.int32)
    one_i = jnp.ones((L,), jnp.int32)

    def chunk_body(c, _):
        s0 = c * SCH
        for g in range(SCH):
            pltpu.sync_copy(ids_hbm.at[s0 + g, pl.ds(p0, PBLK)],
                            idx_v.at[pl.ds(g * PBLK, PBLK)])
            pltpu.sync_copy(tt_hbm.at[s0 + g, pl.ds(p0, PBLK)],
                            tt_v.at[pl.ds(g * PBLK, PBLK)])
        pltpu.async_copy(word_hbm.at[idx_v], rows_v.at[:, pl.ds(0, HIDDEN)],
                         sem).wait()

        for g in range(SCH):
            tb = g * L
            tok_row = lanes + tb
            tt16 = tt_v[pl.ds(tb, L)]

            def p1(i, carry):
                sm, sq, dcol = carry
                for _ in range(UNROLL):
                    v = plsc.load_gather(rows_v, [tok_row, dcol])
                    v = v + plsc.load_gather(pos_v, [lanes, dcol])
                    v = v + plsc.load_gather(type_v, [tt16, dcol])
                    plsc.store_scatter(rows_v, [tok_row, dcol], v)
                    sm = sm + v
                    sq = sq + v * v
                    dcol = dcol + one_i
                return sm, sq, dcol

            sm, sq, _ = lax.fori_loop(0, HIDDEN // UNROLL, p1,
                                      (zero, zero, zero_i))
            mean_v = sm * INV_H
            var_v = sq * INV_H - mean_v * mean_v
            rstd_v = _rsqrt(var_v + EPS)

            for q in range(4):
                mm = [_splat(mean_v, q * 4 + j) for j in range(4)]
                rr = [_splat(rstd_v, q * 4 + j) for j in range(4)]

                def p2(i, _):
                    for u in range(2):
                        sl = pl.ds((i * 2 + u) * L, L)
                        ga = gam_v[sl]
                        be = bet_v[sl]
                        for j in range(4):
                            t = tb + q * 4 + j
                            v = (rows_v[t, sl] - mm[j]) * rr[j]
                            rows_v[t, sl] = v * ga + be
                    return 0

                lax.fori_loop(0, NG // 2, p2, 0)

        for g in range(SCH):
            pltpu.sync_copy(rows_v.at[pl.ds(g * PBLK, PBLK), pl.ds(0, HIDDEN)],
                            out_hbm.at[s0 + g, pl.ds(p0, PBLK), :])
        return 0

    lax.fori_loop(0, NCHUNK, chunk_body, 0)


@jax.jit
def _run(input_ids, token_type_ids, word_table, pos_table, type_table,
         gamma, beta):
    mesh = plsc.VectorSubcoreMesh(core_axis_name="c", subcore_axis_name="s")
    f = pl.kernel(
        _body,
        out_type=jax.ShapeDtypeStruct((BATCH, SEQ, HIDDEN), jnp.float32),
        mesh=mesh,
        compiler_params=pltpu.CompilerParams(needs_layout_passes=False),
        scratch_types=[
            pltpu.VMEM((CHT,), jnp.int32),
            pltpu.VMEM((CHT,), jnp.int32),
            pltpu.VMEM((CHT, PAD), jnp.float32),
            pltpu.VMEM((PBLK, PAD), jnp.float32),
            pltpu.VMEM((TYPE_VOCAB, PAD), jnp.float32),
            pltpu.VMEM((HIDDEN,), jnp.float32),
            pltpu.VMEM((HIDDEN,), jnp.float32),
            pltpu.SemaphoreType.DMA,
        ],
    )
    return f(input_ids, token_type_ids, word_table, pos_table, type_table,
             gamma, beta)


def kernel(input_ids, token_type_ids, word_table, pos_table, type_table,
           gamma, beta):
    return _run(input_ids.astype(jnp.int32), token_type_ids.astype(jnp.int32),
                word_table, pos_table, type_table, gamma, beta)


# speedup vs baseline: 3.6282x; 3.6282x over previous
import functools
import jax
import jax.numpy as jnp
from jax import lax
from jax.experimental import pallas as pl
from jax.experimental.pallas import tpu as pltpu
from jax.experimental.pallas import tpu_sc as plsc

VOCAB = 30522
HIDDEN = 768
MAX_POS = 512
TYPE_VOCAB = 16
BATCH = 128
SEQ = 512

L = 16
NW = 32
PBLK = SEQ // NW
SCH = 4
CHT = SCH * PBLK
NCHUNK = BATCH // SCH
NG = HIDDEN // L
NQ = CHT // 4
INV_H = 1.0 / HIDDEN
EPS = 1e-12


def _rsqrt(x):
    xi = plsc.bitcast(x, jnp.int32)
    yi = jnp.int32(0x5F3759DF) - (xi >> 1)
    y = plsc.bitcast(yi, jnp.float32)
    for _ in range(3):
        y = y * (1.5 - 0.5 * x * y * y)
    return y


def _body(ids_hbm, tt_hbm, word_hbm, pos_hbm, type_hbm, gamma_hbm, beta_hbm,
          out_hbm, idx_v, tt_v, rows_v, pos_v, type_v, gam_v, bet_v, sem):
    wid = lax.axis_index("s") * 2 + lax.axis_index("c")
    p0 = wid * PBLK

    pltpu.sync_copy(pos_hbm.at[pl.ds(p0, PBLK), :], pos_v)
    pltpu.sync_copy(type_hbm, type_v)
    pltpu.sync_copy(gamma_hbm, gam_v)
    pltpu.sync_copy(beta_hbm, bet_v)

    zero = jnp.zeros((L,), jnp.float32)

    def chunk_body(c, _):
        s0 = c * SCH
        for g in range(SCH):
            pltpu.sync_copy(ids_hbm.at[s0 + g, pl.ds(p0, PBLK)],
                            idx_v.at[pl.ds(g * PBLK, PBLK)])
            pltpu.sync_copy(tt_hbm.at[s0 + g, pl.ds(p0, PBLK)],
                            tt_v.at[pl.ds(g * PBLK, PBLK)])
        pltpu.async_copy(word_hbm.at[idx_v], rows_v, sem).wait()

        def group_body(g2, _):
            tb = g2 * L
            tt16 = tt_v[pl.ds(tb, L)]
            for q in range(4):
                mv = []
                rv = []
                for kp in range(2):
                    k0 = q * 4 + kp * 2
                    k1 = k0 + 1
                    t0 = tb + k0
                    t1 = tb + k1
                    tid0 = tt16[k0]
                    tid1 = tt16[k1]

                    def p1(i, carry):
                        sm0, sq0, sm1, sq1 = carry
                        for u in range(8):
                            sl = pl.ds((i * 8 + u) * L, L)
                            v0 = rows_v[t0, sl] + pos_v[k0, sl] \
                                + type_v[tid0, sl]
                            rows_v[t0, sl] = v0
                            v1 = rows_v[t1, sl] + pos_v[k1, sl] \
                                + type_v[tid1, sl]
                            rows_v[t1, sl] = v1
                            sm0 = sm0 + v0
                            sq0 = sq0 + v0 * v0
                            sm1 = sm1 + v1
                            sq1 = sq1 + v1 * v1
                        return sm0, sq0, sm1, sq1

                    sm0, sq0, sm1, sq1 = lax.fori_loop(
                        0, NG // 8, p1, (zero, zero, zero, zero))
                    for sm, sq in ((sm0, sq0), (sm1, sq1)):
                        mean = jnp.sum(sm) * INV_H
                        var = jnp.sum(sq) * INV_H - mean * mean
                        mv.append(jnp.full((L,), mean, jnp.float32))
                        rv.append(_rsqrt(jnp.full((L,), var + EPS,
                                                  jnp.float32)))

                def p2(i, _):
                    for u in range(4):
                        sl = pl.ds((i * 4 + u) * L, L)
                        ga = gam_v[sl]
                        be = bet_v[sl]
                        for k in range(4):
                            t = tb + q * 4 + k
                            v = (rows_v[t, sl] - mv[k]) * rv[k]
                            rows_v[t, sl] = v * ga + be
                    return 0

                lax.fori_loop(0, NG // 4, p2, 0)
            return 0

        lax.fori_loop(0, SCH, group_body, 0)

        for g in range(SCH):
            pltpu.sync_copy(rows_v.at[pl.ds(g * PBLK, PBLK), :],
                            out_hbm.at[s0 + g, pl.ds(p0, PBLK), :])
        return 0

    lax.fori_loop(0, NCHUNK, chunk_body, 0)


@jax.jit
def _run(input_ids, token_type_ids, word_table, pos_table, type_table,
         gamma, beta):
    mesh = plsc.VectorSubcoreMesh(core_axis_name="c", subcore_axis_name="s")
    f = pl.kernel(
        _body,
        out_type=jax.ShapeDtypeStruct((BATCH, SEQ, HIDDEN), jnp.float32),
        mesh=mesh,
        compiler_params=pltpu.CompilerParams(needs_layout_passes=False),
        scratch_types=[
            pltpu.VMEM((CHT,), jnp.int32),
            pltpu.VMEM((CHT,), jnp.int32),
            pltpu.VMEM((CHT, HIDDEN), jnp.float32),
            pltpu.VMEM((PBLK, HIDDEN), jnp.float32),
            pltpu.VMEM((TYPE_VOCAB, HIDDEN), jnp.float32),
            pltpu.VMEM((HIDDEN,), jnp.float32),
            pltpu.VMEM((HIDDEN,), jnp.float32),
            pltpu.SemaphoreType.DMA,
        ],
    )
    return f(input_ids, token_type_ids, word_table, pos_table, type_table,
             gamma, beta)


def kernel(input_ids, token_type_ids, word_table, pos_table, type_table,
           gamma, beta):
    return _run(input_ids.astype(jnp.int32), token_type_ids.astype(jnp.int32),
                word_table, pos_table, type_table, gamma, beta)
